# baseline (speedup 1.0000x reference)
import functools

import jax
import jax.numpy as jnp
from jax.experimental import pallas as pl
from jax.experimental.pallas import tpu as pltpu


def _hgnn_kernel(x_ref, w_ref, b_ref, dv_ref, de_ref, h_ref, out_ref,
                 xn_ref, xm_ref, des_ref, hb_ref, *, num_blocks, block_m):
    i = pl.program_id(0)
    n = x_ref.shape[0]

    @pl.when(i == 0)
    def _prologue():
        xm = jnp.dot(x_ref[...].astype(jnp.bfloat16),
                     w_ref[...].astype(jnp.bfloat16),
                     preferred_element_type=jnp.float32) + b_ref[...]
        xm_ref[...] = xm.astype(jnp.bfloat16)
        xn_ref[...] = (xm * dv_ref[...]).astype(jnp.bfloat16)
        des_ref[...] = jnp.sqrt(de_ref[...])
        hb_ref[pl.ds(n, n), :] = jnp.zeros((n, block_m), jnp.bfloat16)
        out_ref[...] = jnp.zeros_like(out_ref)

    slot = jax.lax.rem(i, 2)
    prev = jax.lax.rem(i + 1, 2)
    jm = jnp.minimum(i, num_blocks - 1)

    des_blk = des_ref[:, pl.ds(jm * block_m, block_m)]
    hb_ref[pl.ds(slot * n, n), :] = (h_ref[...] * des_blk).astype(jnp.bfloat16)

    hbp = hb_ref[pl.ds(prev * n, n), :]
    e2 = jax.lax.dot_general(
        hbp, xn_ref[...],
        dimension_numbers=(((0,), (0,)), ((), ())),
        preferred_element_type=jnp.float32)
    out_ref[...] += jnp.dot(hbp, e2.astype(jnp.bfloat16),
                            preferred_element_type=jnp.float32)

    @pl.when(i == num_blocks)
    def _epilogue():
        out_ref[...] = (out_ref[...] * dv_ref[...]
                        + xm_ref[...].astype(jnp.float32))


@jax.jit
def kernel(x, H, dv_inv, de_inv, weight, bias):
    N, d_in = x.shape
    M = H.shape[1]
    d_out = weight.shape[1]

    Mb = 256
    while M % Mb != 0:
        Mb //= 2
    num_blocks = M // Mb

    dv2 = dv_inv.reshape(N, 1)
    de2 = de_inv.reshape(1, M)
    b2 = bias.reshape(1, d_out)

    out = pl.pallas_call(
        functools.partial(_hgnn_kernel, num_blocks=num_blocks, block_m=Mb),
        grid=(num_blocks + 1,),
        in_specs=[
            pl.BlockSpec((N, d_in), lambda i: (0, 0)),
            pl.BlockSpec((d_in, d_out), lambda i: (0, 0)),
            pl.BlockSpec((1, d_out), lambda i: (0, 0)),
            pl.BlockSpec((N, 1), lambda i: (0, 0)),
            pl.BlockSpec((1, M), lambda i: (0, 0)),
            pl.BlockSpec((N, Mb),
                         lambda i, nb=num_blocks: (0, jnp.minimum(i, nb - 1))),
        ],
        out_specs=pl.BlockSpec((N, d_out), lambda i: (0, 0)),
        out_shape=jax.ShapeDtypeStruct((N, d_out), jnp.float32),
        scratch_shapes=[
            pltpu.VMEM((N, d_out), jnp.bfloat16),
            pltpu.VMEM((N, d_out), jnp.bfloat16),
            pltpu.VMEM((1, M), jnp.float32),
            pltpu.VMEM((2 * N, Mb), jnp.bfloat16),
        ],
        compiler_params=pltpu.CompilerParams(
            dimension_semantics=("arbitrary",),
            vmem_limit_bytes=110 * 1024 * 1024,
        ),
    )(x, weight, b2, dv2, de2, H)
    return out


# speedup vs baseline: 1.4129x; 1.4129x over previous
import functools

import jax
import jax.numpy as jnp
from jax.experimental import pallas as pl
from jax.experimental.pallas import tpu as pltpu


def _hgnn_kernel(x_ref, w_ref, b_ref, dv_ref, de_ref, h_ref, out_ref,
                 xn_ref, xm_ref, *, num_blocks):
    i = pl.program_id(0)

    @pl.when(i == 0)
    def _prologue():
        xm = jnp.dot(x_ref[...].astype(jnp.bfloat16),
                     w_ref[...].astype(jnp.bfloat16),
                     preferred_element_type=jnp.float32) + b_ref[...]
        xm_ref[...] = xm.astype(jnp.bfloat16)
        xn_ref[...] = (xm * dv_ref[...]).astype(jnp.bfloat16)
        out_ref[...] = jnp.zeros_like(out_ref)

    hb = (h_ref[...] * jnp.sqrt(de_ref[...])).astype(jnp.bfloat16)
    e2 = jax.lax.dot_general(
        hb, xn_ref[...],
        dimension_numbers=(((0,), (0,)), ((), ())),
        preferred_element_type=jnp.float32)
    out_ref[...] += jnp.dot(hb, e2.astype(jnp.bfloat16),
                            preferred_element_type=jnp.float32)

    @pl.when(i == num_blocks - 1)
    def _epilogue():
        out_ref[...] = (out_ref[...] * dv_ref[...]
                        + xm_ref[...].astype(jnp.float32))


@jax.jit
def kernel(x, H, dv_inv, de_inv, weight, bias):
    N, d_in = x.shape
    M = H.shape[1]
    d_out = weight.shape[1]

    Mb = 256
    while M % Mb != 0:
        Mb //= 2
    num_blocks = M // Mb

    dv2 = dv_inv.reshape(N, 1)
    de2 = de_inv.reshape(1, M)
    b2 = bias.reshape(1, d_out)

    out = pl.pallas_call(
        functools.partial(_hgnn_kernel, num_blocks=num_blocks),
        grid=(num_blocks,),
        in_specs=[
            pl.BlockSpec((N, d_in), lambda i: (0, 0)),
            pl.BlockSpec((d_in, d_out), lambda i: (0, 0)),
            pl.BlockSpec((1, d_out), lambda i: (0, 0)),
            pl.BlockSpec((N, 1), lambda i: (0, 0)),
            pl.BlockSpec((1, Mb), lambda i: (0, i)),
            pl.BlockSpec((N, Mb), lambda i: (0, i)),
        ],
        out_specs=pl.BlockSpec((N, d_out), lambda i: (0, 0)),
        out_shape=jax.ShapeDtypeStruct((N, d_out), jnp.float32),
        scratch_shapes=[
            pltpu.VMEM((N, d_out), jnp.bfloat16),
            pltpu.VMEM((N, d_out), jnp.bfloat16),
        ],
        compiler_params=pltpu.CompilerParams(
            dimension_semantics=("arbitrary",),
            vmem_limit_bytes=110 * 1024 * 1024,
        ),
    )(x, weight, b2, dv2, de2, H)
    return out
